# baseline (speedup 1.0000x reference)
import functools

import jax
import jax.numpy as jnp
from jax import lax
from jax.experimental import pallas as pl
from jax.experimental.pallas import tpu as pltpu
from jax.experimental.pallas import tpu_sc as plsc

_N = 10000
_NPAD = 10240
_DUMMY = 10100
_EPT = 10240
_NCH = _EPT // 128
_EP = _EPT * 16


def _matmul_body(x_ref, w_ref, o_ref):
    o_ref[...] = jnp.dot(x_ref[...], w_ref[...],
                         preferred_element_type=jnp.float32)


def _matmul(x, w):
    npad, f = x.shape
    c = w.shape[1]
    nb = npad // 256
    return pl.pallas_call(
        _matmul_body,
        grid=(nb,),
        in_specs=[
            pl.BlockSpec((256, f), lambda i: (i, 0)),
            pl.BlockSpec((f, c), lambda i: (0, 0)),
        ],
        out_specs=pl.BlockSpec((256, c), lambda i: (i, 0)),
        out_shape=jax.ShapeDtypeStruct((npad, c), jnp.float32),
    )(x, w)


def _sc_aggregate(ytab, gidx, dste):
    mesh = plsc.VectorSubcoreMesh(core_axis_name="c", subcore_axis_name="s")
    zrows = jnp.zeros((640, 128), jnp.float32)

    @functools.partial(
        pl.kernel, mesh=mesh,
        out_type=jax.ShapeDtypeStruct((2 * _NPAD, 128), jnp.float32),
        scratch_types=[
            pltpu.VMEM((128,), jnp.int32),
            pltpu.VMEM((128,), jnp.int32),
            pltpu.VMEM((128,), jnp.int32),
            pltpu.VMEM((128,), jnp.int32),
            pltpu.VMEM((128, 128), jnp.float32),
            pltpu.VMEM((128, 128), jnp.float32),
            pltpu.SemaphoreType.DMA,
            pltpu.SemaphoreType.DMA,
            pltpu.VMEM_SHARED((_NPAD, 128), jnp.float32),
        ],
    )
    def k(gidx_h, dst_h, zrows_h, ytab_h, agg_o,
          idx0_v, idx1_v, dst0_v, dst1_v, rows0_v, rows1_v, sem0, sem1,
          agg_s):
        c = lax.axis_index("c")
        s = lax.axis_index("s")
        rbase = s * 640
        pltpu.sync_copy(zrows_h, agg_s.at[pl.ds(rbase, 640)])
        plsc.subcore_barrier()

        ebase = s * _EPT
        gbase = c * _EP + ebase

        def start(j, idx_v, dst_v, rows_v, sem):
            go = pl.multiple_of(gbase + j * 128, 128)
            eo = pl.multiple_of(ebase + j * 128, 128)
            pltpu.sync_copy(gidx_h.at[pl.ds(go, 128)], idx_v)
            pltpu.sync_copy(dst_h.at[pl.ds(eo, 128)], dst_v)
            pltpu.async_copy(ytab_h.at[idx_v], rows_v, sem)

        def finish(dst_v, rows_v, sem):
            pltpu.make_async_copy(ytab_h.at[pl.ds(0, 128)], rows_v, sem).wait()
            pltpu.sync_copy(rows_v, agg_s.at[dst_v], add=True)

        start(0, idx0_v, dst0_v, rows0_v, sem0)

        def pair(i, carry):
            start(2 * i + 1, idx1_v, dst1_v, rows1_v, sem1)
            finish(dst0_v, rows0_v, sem0)
            j2 = jnp.where(i == _NCH // 2 - 1, 0, 2 * i + 2)
            start(j2, idx0_v, dst0_v, rows0_v, sem0)
            finish(dst1_v, rows1_v, sem1)
            return carry

        lax.fori_loop(0, _NCH // 2, pair, 0)
        pltpu.make_async_copy(ytab_h.at[pl.ds(0, 128)], rows0_v, sem0).wait()
        plsc.subcore_barrier()

        obase = pl.multiple_of(c * _NPAD + rbase, 128)
        pltpu.sync_copy(agg_s.at[pl.ds(rbase, 640)], agg_o.at[pl.ds(obase, 640)])

    return k(gidx, dste, zrows, ytab)


_EPT2 = 5120
_NCH2 = _EPT2 // 128
_EP2 = _EPT2 * 32


def _sc_degree(dste2):
    mesh = plsc.VectorSubcoreMesh(core_axis_name="c", subcore_axis_name="s")
    zrows = jnp.zeros((640, 128), jnp.float32)
    ones = jnp.ones((128, 128), jnp.float32)

    @functools.partial(
        pl.kernel, mesh=mesh,
        out_type=jax.ShapeDtypeStruct((2 * _NPAD, 128), jnp.float32),
        scratch_types=[
            pltpu.VMEM((128,), jnp.int32),
            pltpu.VMEM((128, 128), jnp.float32),
            pltpu.VMEM_SHARED((_NPAD, 128), jnp.float32),
        ],
    )
    def k(dst_h, zrows_h, ones_h, deg_o, dst_v, ones_v, deg_s):
        c = lax.axis_index("c")
        s = lax.axis_index("s")
        rbase = s * 640
        pltpu.sync_copy(zrows_h, deg_s.at[pl.ds(rbase, 640)])
        pltpu.sync_copy(ones_h, ones_v)
        plsc.subcore_barrier()

        ebase = (s * 2 + c) * _EPT2

        def chunk(j, carry):
            eo = pl.multiple_of(ebase + j * 128, 128)
            pltpu.sync_copy(dst_h.at[pl.ds(eo, 128)], dst_v)
            pltpu.sync_copy(ones_v, deg_s.at[dst_v], add=True)
            return carry

        lax.fori_loop(0, _NCH2, chunk, 0)
        plsc.subcore_barrier()

        obase = pl.multiple_of(c * _NPAD + rbase, 128)
        pltpu.sync_copy(deg_s.at[pl.ds(rbase, 640)], deg_o.at[pl.ds(obase, 640)])

    return k(dste2, zrows, ones)


_QPT = 256
_Q = _QPT * 32


def _sc_decode_gather(embp, rtab, srcq, dstq, relq):
    mesh = plsc.VectorSubcoreMesh(core_axis_name="c", subcore_axis_name="s")
    otype = jax.ShapeDtypeStruct((_Q, 256), jnp.float32)

    @functools.partial(
        pl.kernel, mesh=mesh,
        out_type=[otype, otype, otype],
        scratch_types=[
            pltpu.VMEM((128,), jnp.int32),
            pltpu.VMEM((128, 256), jnp.float32),
            pltpu.SemaphoreType.DMA,
        ],
    )
    def k(src_h, dst_h, rel_h, emb_h, rtab_h, es_o, ed_o, er_o,
          idx_v, rows_v, sem):
        c = lax.axis_index("c")
        s = lax.axis_index("s")
        qbase = (s * 2 + c) * _QPT

        def chunk(j, carry):
            qo = pl.multiple_of(qbase + j * 128, 128)
            for ih, th, oh in ((src_h, emb_h, es_o), (dst_h, emb_h, ed_o),
                               (rel_h, rtab_h, er_o)):
                pltpu.sync_copy(ih.at[pl.ds(qo, 128)], idx_v)
                pltpu.async_copy(th.at[idx_v], rows_v, sem).wait()
                pltpu.sync_copy(rows_v, oh.at[pl.ds(qo, 128)])
            return carry

        lax.fori_loop(0, _QPT // 128, chunk, 0)

    return k(srcq, dstq, relq, embp, rtab)


def _score_body(es_ref, er_ref, ed_ref, o_ref):
    s = jnp.sum(es_ref[...] * er_ref[...] * ed_ref[...], axis=1)
    o_ref[...] = s.reshape(o_ref.shape)


def _distmult_score(es, er, ed):
    grid = _Q // 1024
    out = pl.pallas_call(
        _score_body,
        grid=(grid,),
        in_specs=[pl.BlockSpec((1024, 256), lambda i: (i, 0))] * 3,
        out_specs=pl.BlockSpec((8, 128), lambda i: (i, 0)),
        out_shape=jax.ShapeDtypeStruct((_Q // 128, 128), jnp.float32),
    )(es, er, ed)
    return out.reshape(_Q)


def _mixw_body(nb, c_ref, b_ref, o_ref):
    r = pl.program_id(0)
    acc = b_ref[0] * c_ref[r, 0]
    for b in range(1, nb):
        acc += b_ref[b] * c_ref[r, b]
    o_ref[...] = acc


def _mix_weights(coeffs, bases):
    r, nb = coeffs.shape
    _, f, h = bases.shape
    return pl.pallas_call(
        functools.partial(_mixw_body, nb),
        grid=(r,),
        in_specs=[
            pl.BlockSpec(memory_space=pltpu.SMEM),
            pl.BlockSpec((nb, f, h), lambda i: (0, 0, 0)),
        ],
        out_specs=pl.BlockSpec((f, h), lambda i: (0, i)),
        out_shape=jax.ShapeDtypeStruct((f, r * h), jnp.float32),
    )(coeffs, bases)


def _combine_body(relu, agg_ref, deg_ref, selfb_ref, bias_ref, o_ref):
    deg = jnp.maximum(deg_ref[0, 0, :], 1.0)
    out = agg_ref[0] / deg[:, None] + selfb_ref[...] + bias_ref[0, 0]
    if relu:
        out = jnp.maximum(out, 0.0)
    o_ref[...] = out


def _combine(aggs, deg2, selfb, bias, relu):
    nb = _NPAD // 128
    return pl.pallas_call(
        functools.partial(_combine_body, relu),
        grid=(nb, 2),
        in_specs=[
            pl.BlockSpec((1, 128, 128), lambda i, c: (c, i, 0)),
            pl.BlockSpec((1, 1, 128), lambda i, c: (i, 0, 0)),
            pl.BlockSpec((128, 128), lambda i, c: (i, c)),
            pl.BlockSpec((1, 1, 128), lambda i, c: (c, 0, 0)),
        ],
        out_specs=pl.BlockSpec((128, 128), lambda i, c: (i, c)),
        out_shape=jax.ShapeDtypeStruct((_NPAD, 256), jnp.float32),
    )(aggs, deg2, selfb, bias.reshape(2, 1, 128))


def _layer(xp, gidx, dste, deg2, wcat, wself, bias, relu):
    y = _matmul(xp, wcat)
    selfb = _matmul(xp, wself)
    ytab = y.reshape(_NPAD * (y.shape[1] // 128), 128)
    agg = _sc_aggregate(ytab, gidx, dste)
    return _combine(agg.reshape(2, _NPAD, 128), deg2, selfb, bias, relu)


def kernel(edge_index, edge_type, src, dst, rel, node_features, num_nodes,
           bases0, coeffs0, wself0, bias0,
           bases1, coeffs1, wself1, bias1,
           rel_emb, wcls, bcls):
    n, f = node_features.shape
    r = coeffs0.shape[0]
    h = wself0.shape[1]

    g0 = edge_index[0] * (2 * r) + edge_type * 2
    pad = _EP - g0.shape[0]
    g0p = jnp.pad(g0, (0, pad))
    gidx = jnp.concatenate([g0p, g0p + 1]).astype(jnp.int32)
    dste = jnp.pad(edge_index[1], (0, pad),
                   constant_values=_DUMMY).astype(jnp.int32)

    degp = _sc_degree(dste)
    deg2 = (degp[:_NPAD, 0] + degp[_NPAD:, 0]).reshape(_NPAD // 128, 1, 128)

    xp = jnp.pad(node_features, ((0, _NPAD - n), (0, 0)))
    w0 = _mix_weights(coeffs0, bases0)
    h1 = _layer(xp, gidx, dste, deg2, w0, wself0, bias0, relu=True)
    w1 = _mix_weights(coeffs1, bases1)
    emb_p = _layer(h1, gidx, dste, deg2, w1, wself1, bias1, relu=False)

    es, ed, er = _sc_decode_gather(emb_p, rel_emb,
                                   src.astype(jnp.int32),
                                   dst.astype(jnp.int32),
                                   rel.astype(jnp.int32))
    link_scores = _distmult_score(es, er, ed)
    wcls_p = jnp.pad(wcls, ((0, 0), (0, 128 - wcls.shape[1])))
    node_logits = _matmul(emb_p, wcls_p)[:n, :wcls.shape[1]] + bcls
    return (link_scores, node_logits)


# speedup vs baseline: 1.0575x; 1.0575x over previous
import functools

import jax
import jax.numpy as jnp
from jax import lax
from jax.experimental import pallas as pl
from jax.experimental.pallas import tpu as pltpu
from jax.experimental.pallas import tpu_sc as plsc

_N = 10000
_NPAD = 10240
_DUMMY = 10100
_EPT = 10240
_NCH = _EPT // 128
_EP = _EPT * 16


def _matmul_body(x_ref, w_ref, o_ref):
    o_ref[...] = jnp.dot(x_ref[...], w_ref[...],
                         preferred_element_type=jnp.float32)


def _matmul(x, w):
    npad, f = x.shape
    c = w.shape[1]
    nb = npad // 256
    return pl.pallas_call(
        _matmul_body,
        grid=(nb,),
        in_specs=[
            pl.BlockSpec((256, f), lambda i: (i, 0)),
            pl.BlockSpec((f, c), lambda i: (0, 0)),
        ],
        out_specs=pl.BlockSpec((256, c), lambda i: (i, 0)),
        out_shape=jax.ShapeDtypeStruct((npad, c), jnp.float32),
    )(x, w)


def _sc_aggregate(ytab, gidx, dste):
    mesh = plsc.VectorSubcoreMesh(core_axis_name="c", subcore_axis_name="s")
    zrows = jnp.zeros((640, 128), jnp.float32)

    @functools.partial(
        pl.kernel, mesh=mesh,
        out_type=jax.ShapeDtypeStruct((2 * _NPAD, 128), jnp.float32),
        scratch_types=[
            pltpu.VMEM((128,), jnp.int32),
            pltpu.VMEM((128,), jnp.int32),
            pltpu.VMEM((128,), jnp.int32),
            pltpu.VMEM((128,), jnp.int32),
            pltpu.VMEM((128, 128), jnp.float32),
            pltpu.VMEM((128, 128), jnp.float32),
            pltpu.SemaphoreType.DMA,
            pltpu.SemaphoreType.DMA,
            pltpu.VMEM_SHARED((_NPAD, 128), jnp.float32),
        ],
    )
    def k(gidx_h, dst_h, zrows_h, ytab_h, agg_o,
          idx0_v, idx1_v, dst0_v, dst1_v, rows0_v, rows1_v, sem0, sem1,
          agg_s):
        c = lax.axis_index("c")
        s = lax.axis_index("s")
        rbase = s * 640
        pltpu.sync_copy(zrows_h, agg_s.at[pl.ds(rbase, 640)])
        plsc.subcore_barrier()

        ebase = s * _EPT
        gbase = c * _EP + ebase

        def start(j, idx_v, dst_v, rows_v, sem):
            go = pl.multiple_of(gbase + j * 128, 128)
            eo = pl.multiple_of(ebase + j * 128, 128)
            pltpu.sync_copy(gidx_h.at[pl.ds(go, 128)], idx_v)
            pltpu.sync_copy(dst_h.at[pl.ds(eo, 128)], dst_v)
            pltpu.async_copy(ytab_h.at[idx_v], rows_v, sem)

        def finish(dst_v, rows_v, sem):
            pltpu.make_async_copy(ytab_h.at[pl.ds(0, 128)], rows_v, sem).wait()
            pltpu.sync_copy(rows_v, agg_s.at[dst_v], add=True)

        start(0, idx0_v, dst0_v, rows0_v, sem0)

        def pair(i, carry):
            start(2 * i + 1, idx1_v, dst1_v, rows1_v, sem1)
            finish(dst0_v, rows0_v, sem0)
            j2 = jnp.where(i == _NCH // 2 - 1, 0, 2 * i + 2)
            start(j2, idx0_v, dst0_v, rows0_v, sem0)
            finish(dst1_v, rows1_v, sem1)
            return carry

        lax.fori_loop(0, _NCH // 2, pair, 0)
        pltpu.make_async_copy(ytab_h.at[pl.ds(0, 128)], rows0_v, sem0).wait()
        plsc.subcore_barrier()

        obase = pl.multiple_of(c * _NPAD + rbase, 128)
        pltpu.sync_copy(agg_s.at[pl.ds(rbase, 640)], agg_o.at[pl.ds(obase, 640)])

    return k(gidx, dste, zrows, ytab)


_EPT2 = 5120
_NCH2 = _EPT2 // 128
_EP2 = _EPT2 * 32


def _sc_degree(dste2):
    mesh = plsc.VectorSubcoreMesh(core_axis_name="c", subcore_axis_name="s")
    zrows = jnp.zeros((640, 128), jnp.float32)
    ones = jnp.ones((128, 128), jnp.float32)

    @functools.partial(
        pl.kernel, mesh=mesh,
        out_type=jax.ShapeDtypeStruct((2 * _NPAD, 128), jnp.float32),
        scratch_types=[
            pltpu.VMEM((128,), jnp.int32),
            pltpu.VMEM((128, 128), jnp.float32),
            pltpu.VMEM_SHARED((_NPAD, 128), jnp.float32),
        ],
    )
    def k(dst_h, zrows_h, ones_h, deg_o, dst_v, ones_v, deg_s):
        c = lax.axis_index("c")
        s = lax.axis_index("s")
        rbase = s * 640
        pltpu.sync_copy(zrows_h, deg_s.at[pl.ds(rbase, 640)])
        pltpu.sync_copy(ones_h, ones_v)
        plsc.subcore_barrier()

        ebase = (s * 2 + c) * _EPT2

        def chunk(j, carry):
            eo = pl.multiple_of(ebase + j * 128, 128)
            pltpu.sync_copy(dst_h.at[pl.ds(eo, 128)], dst_v)
            pltpu.sync_copy(ones_v, deg_s.at[dst_v], add=True)
            return carry

        lax.fori_loop(0, _NCH2, chunk, 0)
        plsc.subcore_barrier()

        obase = pl.multiple_of(c * _NPAD + rbase, 128)
        pltpu.sync_copy(deg_s.at[pl.ds(rbase, 640)], deg_o.at[pl.ds(obase, 640)])

    return k(dste2, zrows, ones)


_QPT = 256
_Q = _QPT * 32


def _sc_decode_gather(embp, rtab, srcq, dstq, relq):
    mesh = plsc.VectorSubcoreMesh(core_axis_name="c", subcore_axis_name="s")
    otype = jax.ShapeDtypeStruct((_Q, 256), jnp.float32)

    @functools.partial(
        pl.kernel, mesh=mesh,
        out_type=[otype, otype, otype],
        scratch_types=[
            pltpu.VMEM((128,), jnp.int32),
            pltpu.VMEM((128, 256), jnp.float32),
            pltpu.SemaphoreType.DMA,
        ],
    )
    def k(src_h, dst_h, rel_h, emb_h, rtab_h, es_o, ed_o, er_o,
          idx_v, rows_v, sem):
        c = lax.axis_index("c")
        s = lax.axis_index("s")
        qbase = (s * 2 + c) * _QPT

        def chunk(j, carry):
            qo = pl.multiple_of(qbase + j * 128, 128)
            for ih, th, oh in ((src_h, emb_h, es_o), (dst_h, emb_h, ed_o),
                               (rel_h, rtab_h, er_o)):
                pltpu.sync_copy(ih.at[pl.ds(qo, 128)], idx_v)
                pltpu.async_copy(th.at[idx_v], rows_v, sem).wait()
                pltpu.sync_copy(rows_v, oh.at[pl.ds(qo, 128)])
            return carry

        lax.fori_loop(0, _QPT // 128, chunk, 0)

    return k(srcq, dstq, relq, embp, rtab)


def _score_body(es_ref, er_ref, ed_ref, o_ref):
    s = jnp.sum(es_ref[...] * er_ref[...] * ed_ref[...], axis=1)
    o_ref[...] = s.reshape(o_ref.shape)


def _distmult_score(es, er, ed):
    grid = _Q // 1024
    out = pl.pallas_call(
        _score_body,
        grid=(grid,),
        in_specs=[pl.BlockSpec((1024, 256), lambda i: (i, 0))] * 3,
        out_specs=pl.BlockSpec((8, 128), lambda i: (i, 0)),
        out_shape=jax.ShapeDtypeStruct((_Q // 128, 128), jnp.float32),
    )(es, er, ed)
    return out.reshape(_Q)


def _mixw_body(nb, c_ref, b_ref, o_ref):
    r = pl.program_id(0)
    acc = b_ref[0] * c_ref[r, 0]
    for b in range(1, nb):
        acc += b_ref[b] * c_ref[r, b]
    o_ref[...] = acc


def _mix_weights(coeffs, bases):
    r, nb = coeffs.shape
    _, f, h = bases.shape
    return pl.pallas_call(
        functools.partial(_mixw_body, nb),
        grid=(r,),
        in_specs=[
            pl.BlockSpec(memory_space=pltpu.SMEM),
            pl.BlockSpec((nb, f, h), lambda i: (0, 0, 0)),
        ],
        out_specs=pl.BlockSpec((f, h), lambda i: (0, i)),
        out_shape=jax.ShapeDtypeStruct((f, r * h), jnp.float32),
    )(coeffs, bases)


def _combine_body(relu, agg_ref, deg_ref, selfb_ref, bias_ref, o_ref):
    deg = jnp.maximum(deg_ref[0, 0, :], 1.0)
    out = agg_ref[0] / deg[:, None] + selfb_ref[...] + bias_ref[0, 0]
    if relu:
        out = jnp.maximum(out, 0.0)
    o_ref[...] = out


def _combine(aggs, deg2, z, bias, relu):
    nb = _NPAD // 128
    off = z.shape[1] // 128 - 2
    return pl.pallas_call(
        functools.partial(_combine_body, relu),
        grid=(nb, 2),
        in_specs=[
            pl.BlockSpec((1, 128, 128), lambda i, c: (c, i, 0)),
            pl.BlockSpec((1, 1, 128), lambda i, c: (i, 0, 0)),
            pl.BlockSpec((128, 128), lambda i, c: (i, off + c)),
            pl.BlockSpec((1, 1, 128), lambda i, c: (c, 0, 0)),
        ],
        out_specs=pl.BlockSpec((128, 128), lambda i, c: (i, c)),
        out_shape=jax.ShapeDtypeStruct((_NPAD, 256), jnp.float32),
    )(aggs, deg2, z, bias.reshape(2, 1, 128))


def _layer(xp, gidx, dste, deg2, wcat, wself, bias, relu):
    z = _matmul(xp, jnp.concatenate([wcat, wself], axis=1))
    ytab = z.reshape(_NPAD * (z.shape[1] // 128), 128)
    agg = _sc_aggregate(ytab, gidx, dste)
    return _combine(agg.reshape(2, _NPAD, 128), deg2, z, bias, relu)


def kernel(edge_index, edge_type, src, dst, rel, node_features, num_nodes,
           bases0, coeffs0, wself0, bias0,
           bases1, coeffs1, wself1, bias1,
           rel_emb, wcls, bcls):
    n, f = node_features.shape
    r = coeffs0.shape[0]
    h = wself0.shape[1]

    g0 = edge_index[0] * (2 * r + 2) + edge_type * 2
    pad = _EP - g0.shape[0]
    g0p = jnp.pad(g0, (0, pad))
    gidx = jnp.concatenate([g0p, g0p + 1]).astype(jnp.int32)
    dste = jnp.pad(edge_index[1], (0, pad),
                   constant_values=_DUMMY).astype(jnp.int32)

    degp = _sc_degree(dste)
    deg2 = (degp[:_NPAD, 0] + degp[_NPAD:, 0]).reshape(_NPAD // 128, 1, 128)

    xp = jnp.pad(node_features, ((0, _NPAD - n), (0, 0)))
    w0 = _mix_weights(coeffs0, bases0)
    h1 = _layer(xp, gidx, dste, deg2, w0, wself0, bias0, relu=True)
    w1 = _mix_weights(coeffs1, bases1)
    emb_p = _layer(h1, gidx, dste, deg2, w1, wself1, bias1, relu=False)

    es, ed, er = _sc_decode_gather(emb_p, rel_emb,
                                   src.astype(jnp.int32),
                                   dst.astype(jnp.int32),
                                   rel.astype(jnp.int32))
    link_scores = _distmult_score(es, er, ed)
    wcls_p = jnp.pad(wcls, ((0, 0), (0, 128 - wcls.shape[1])))
    node_logits = _matmul(emb_p, wcls_p)[:n, :wcls.shape[1]] + bcls
    return (link_scores, node_logits)
